# Initial kernel scaffold; baseline (speedup 1.0000x reference)
import functools

import jax
import jax.numpy as jnp
from jax import lax
from jax.experimental import pallas as pl
from jax.experimental.pallas import tpu as pltpu
from jax.experimental.pallas import tpu_sc as plsc

N = 10000
E = 320000
D = 128
H = 128
C = 40
CP = 64

NC = 2
NS = 16
NW = NC * NS
CH = 80
EPW = E // NW
NCHUNK = EPW // CH
RPT = N // NS

_mesh = plsc.VectorSubcoreMesh(core_axis_name="c", subcore_axis_name="s")


@functools.partial(
    pl.kernel,
    out_type=[
        jax.ShapeDtypeStruct((NC, N, 8), jnp.float32),
        jax.ShapeDtypeStruct((NC, N, 8), jnp.float32),
    ],
    mesh=_mesh,
    scratch_types=[
        pltpu.VMEM((NCHUNK, CH), jnp.int32),
        pltpu.VMEM((NCHUNK, CH), jnp.int32),
        pltpu.VMEM((CH, 8), jnp.float32),
        pltpu.VMEM_SHARED((N, 8), jnp.float32),
        pltpu.VMEM_SHARED((N, 8), jnp.float32),
    ],
)
def _deg_kernel(dst1_hbm, dst2_hbm, ones_hbm, zeros_hbm, out1_hbm, out2_hbm,
                d1_v, d2_v, ones_v, acc1_sh, acc2_sh):
    c = lax.axis_index("c")
    s = lax.axis_index("s")
    wid = s * NC + c
    pltpu.sync_copy(zeros_hbm.at[pl.ds(s * RPT, RPT)], acc1_sh.at[pl.ds(s * RPT, RPT)])
    pltpu.sync_copy(zeros_hbm.at[pl.ds(s * RPT, RPT)], acc2_sh.at[pl.ds(s * RPT, RPT)])
    pltpu.sync_copy(dst1_hbm.at[wid], d1_v)
    pltpu.sync_copy(dst2_hbm.at[wid], d2_v)
    pltpu.sync_copy(ones_hbm, ones_v)
    plsc.subcore_barrier()

    def body(j, carry):
        pltpu.sync_copy(ones_v, acc1_sh.at[d1_v.at[j]], add=True)
        pltpu.sync_copy(ones_v, acc2_sh.at[d2_v.at[j]], add=True)
        return carry

    lax.fori_loop(0, NCHUNK, body, 0)
    plsc.subcore_barrier()
    pltpu.sync_copy(acc1_sh.at[pl.ds(s * RPT, RPT)], out1_hbm.at[c, pl.ds(s * RPT, RPT)])
    pltpu.sync_copy(acc2_sh.at[pl.ds(s * RPT, RPT)], out2_hbm.at[c, pl.ds(s * RPT, RPT)])


def _make_agg(F):
    @functools.partial(
        pl.kernel,
        out_type=jax.ShapeDtypeStruct((NC, N, F), jnp.float32),
        mesh=_mesh,
        scratch_types=[
            pltpu.VMEM((NCHUNK, CH), jnp.int32),
            pltpu.VMEM((NCHUNK, CH), jnp.int32),
            pltpu.VMEM((CH, F), jnp.float32),
            pltpu.VMEM_SHARED((N, F), jnp.float32),
            pltpu.SemaphoreType.DMA,
        ],
    )
    def agg(g_hbm, src_hbm, dst_hbm, zeros_hbm, out_hbm, src_v, dst_v, buf_v,
            acc_sh, sem):
        c = lax.axis_index("c")
        s = lax.axis_index("s")
        wid = s * NC + c
        pltpu.sync_copy(zeros_hbm.at[pl.ds(s * RPT, RPT)], acc_sh.at[pl.ds(s * RPT, RPT)])
        pltpu.sync_copy(src_hbm.at[wid], src_v)
        pltpu.sync_copy(dst_hbm.at[wid], dst_v)
        plsc.subcore_barrier()

        def body(j, carry):
            pltpu.async_copy(g_hbm.at[src_v.at[j]], buf_v, sem).wait()
            pltpu.sync_copy(buf_v, acc_sh.at[dst_v.at[j]], add=True)
            return carry

        lax.fori_loop(0, NCHUNK, body, 0)
        plsc.subcore_barrier()
        pltpu.sync_copy(acc_sh.at[pl.ds(s * RPT, RPT)], out_hbm.at[c, pl.ds(s * RPT, RPT)])

    return agg


_agg_h = _make_agg(H)
_agg_c = _make_agg(CP)


def _stage_b_body(x1_ref, x2_ref, w1_ref, degcat_ref, g1_ref, g2_ref):
    dc = degcat_ref[...]
    di1 = lax.rsqrt(dc[:, 0:1] + dc[:, 8:9] + 1.0)
    di2 = lax.rsqrt(dc[:, 16:17] + dc[:, 24:25] + 1.0)
    w1 = w1_ref[...]
    g1_ref[...] = jnp.dot(x1_ref[...], w1, preferred_element_type=jnp.float32) * di1
    g2_ref[...] = jnp.dot(x2_ref[...], w1, preferred_element_type=jnp.float32) * di2


_BR = 1000


def _stage_b(x1, x2, W1, degcat):
    grid = N // _BR
    return pl.pallas_call(
        _stage_b_body,
        grid=(grid,),
        in_specs=[
            pl.BlockSpec((_BR, D), lambda i: (i, 0)),
            pl.BlockSpec((_BR, D), lambda i: (i, 0)),
            pl.BlockSpec((D, H), lambda i: (0, 0)),
            pl.BlockSpec((_BR, 32), lambda i: (i, 0)),
        ],
        out_specs=[
            pl.BlockSpec((_BR, H), lambda i: (i, 0)),
            pl.BlockSpec((_BR, H), lambda i: (i, 0)),
        ],
        out_shape=[
            jax.ShapeDtypeStruct((N, H), jnp.float32),
            jax.ShapeDtypeStruct((N, H), jnp.float32),
        ],
    )(x1, x2, W1, degcat)


def _stage_d_body(g1_ref, a10_ref, a11_ref, g2_ref, a20_ref, a21_ref,
                  w2_ref, b1_ref, degcat_ref, gy1_ref, gy2_ref):
    dc = degcat_ref[...]
    di1 = lax.rsqrt(dc[:, 0:1] + dc[:, 8:9] + 1.0)
    di2 = lax.rsqrt(dc[:, 16:17] + dc[:, 24:25] + 1.0)
    w2 = w2_ref[...]
    b1 = b1_ref[...]
    h1 = jnp.maximum(di1 * (a10_ref[...] + a11_ref[...] + g1_ref[...]) + b1, 0.0)
    h2 = jnp.maximum(di2 * (a20_ref[...] + a21_ref[...] + g2_ref[...]) + b1, 0.0)
    gy1_ref[...] = jnp.dot(h1, w2, preferred_element_type=jnp.float32) * di1
    gy2_ref[...] = jnp.dot(h2, w2, preferred_element_type=jnp.float32) * di2


def _stage_d(g1, a10, a11, g2, a20, a21, W2p, b1r, degcat):
    grid = N // _BR
    row = lambda i: (i, 0)
    fixed = lambda i: (0, 0)
    return pl.pallas_call(
        _stage_d_body,
        grid=(grid,),
        in_specs=[
            pl.BlockSpec((_BR, H), row),
            pl.BlockSpec((_BR, H), row),
            pl.BlockSpec((_BR, H), row),
            pl.BlockSpec((_BR, H), row),
            pl.BlockSpec((_BR, H), row),
            pl.BlockSpec((_BR, H), row),
            pl.BlockSpec((H, CP), fixed),
            pl.BlockSpec((1, H), fixed),
            pl.BlockSpec((_BR, 32), row),
        ],
        out_specs=[
            pl.BlockSpec((_BR, CP), row),
            pl.BlockSpec((_BR, CP), row),
        ],
        out_shape=[
            jax.ShapeDtypeStruct((N, CP), jnp.float32),
            jax.ShapeDtypeStruct((N, CP), jnp.float32),
        ],
    )(g1, a10, a11, g2, a20, a21, W2p, b1r, degcat)


def _stage_f_body(gy1_ref, ay10_ref, ay11_ref, gy2_ref, ay20_ref, ay21_ref,
                  b2_ref, degcat_ref, ly_ref, lz_ref, cd_ref):
    dc = degcat_ref[...]
    di1 = lax.rsqrt(dc[:, 0:1] + dc[:, 8:9] + 1.0)
    di2 = lax.rsqrt(dc[:, 16:17] + dc[:, 24:25] + 1.0)
    b2 = b2_ref[...]
    y = di1 * (ay10_ref[...] + ay11_ref[...] + gy1_ref[...]) + b2
    z = di2 * (ay20_ref[...] + ay21_ref[...] + gy2_ref[...]) + b2
    ny = jnp.maximum(jnp.sqrt(jnp.sum(y * y, axis=1, keepdims=True)), 1e-8)
    nz = jnp.maximum(jnp.sqrt(jnp.sum(z * z, axis=1, keepdims=True)), 1e-8)
    cos = jnp.sum(y * z, axis=1, keepdims=True) / (ny * nz)
    cd_ref[...] = 1.0 - cos
    mask = lax.broadcasted_iota(jnp.int32, (_BR, CP), 1) < C
    neg = jnp.float32(-1e30)
    my = jnp.max(jnp.where(mask, y, neg), axis=1, keepdims=True)
    mz = jnp.max(jnp.where(mask, z, neg), axis=1, keepdims=True)
    lse_y = jnp.log(jnp.sum(jnp.where(mask, jnp.exp(y - my), 0.0), axis=1, keepdims=True))
    lse_z = jnp.log(jnp.sum(jnp.where(mask, jnp.exp(z - mz), 0.0), axis=1, keepdims=True))
    ly_ref[...] = y - my - lse_y
    lz_ref[...] = z - mz - lse_z


def _stage_f(gy1, ay10, ay11, gy2, ay20, ay21, b2p, degcat):
    grid = N // _BR
    row = lambda i: (i, 0)
    fixed = lambda i: (0, 0)
    return pl.pallas_call(
        _stage_f_body,
        grid=(grid,),
        in_specs=[
            pl.BlockSpec((_BR, CP), row),
            pl.BlockSpec((_BR, CP), row),
            pl.BlockSpec((_BR, CP), row),
            pl.BlockSpec((_BR, CP), row),
            pl.BlockSpec((_BR, CP), row),
            pl.BlockSpec((_BR, CP), row),
            pl.BlockSpec((1, CP), fixed),
            pl.BlockSpec((_BR, 32), row),
        ],
        out_specs=[
            pl.BlockSpec((_BR, CP), row),
            pl.BlockSpec((_BR, CP), row),
            pl.BlockSpec((_BR, 1), row),
        ],
        out_shape=[
            jax.ShapeDtypeStruct((N, CP), jnp.float32),
            jax.ShapeDtypeStruct((N, CP), jnp.float32),
            jax.ShapeDtypeStruct((N, 1), jnp.float32),
        ],
    )(gy1, ay10, ay11, gy2, ay20, ay21, b2p, degcat)


def kernel(x1, edge_index1, x2, edge_index2, W1, b1, W2, b2):
    src1 = edge_index1[0].reshape(NW, NCHUNK, CH)
    dst1 = edge_index1[1].reshape(NW, NCHUNK, CH)
    src2 = edge_index2[0].reshape(NW, NCHUNK, CH)
    dst2 = edge_index2[1].reshape(NW, NCHUNK, CH)

    ones8 = jnp.ones((CH, 8), jnp.float32)
    zeros8 = jnp.zeros((N, 8), jnp.float32)
    zeros_h = jnp.zeros((N, H), jnp.float32)
    zeros_c = jnp.zeros((N, CP), jnp.float32)

    W2p = jnp.zeros((H, CP), jnp.float32).at[:, :C].set(W2)
    b1r = b1.reshape(1, H)
    b2p = jnp.zeros((1, CP), jnp.float32).at[0, :C].set(b2)

    deg1p, deg2p = _deg_kernel(dst1, dst2, ones8, zeros8)
    degcat = jnp.concatenate([deg1p[0], deg1p[1], deg2p[0], deg2p[1]], axis=1)

    g1, g2 = _stage_b(x1, x2, W1, degcat)
    a1 = _agg_h(g1, src1, dst1, zeros_h)
    a2 = _agg_h(g2, src2, dst2, zeros_h)
    gy1, gy2 = _stage_d(g1, a1[0], a1[1], g2, a2[0], a2[1], W2p, b1r, degcat)
    ay1 = _agg_c(gy1, src1, dst1, zeros_c)
    ay2 = _agg_c(gy2, src2, dst2, zeros_c)
    ly64, lz64, cd = _stage_f(gy1, ay1[0], ay1[1], gy2, ay2[0], ay2[1], b2p, degcat)

    ly = ly64[:, :C]
    lz = lz64[:, :C]
    return (ly, cd[:, 0], lz, ly, ly)


# speedup vs baseline: 16.4700x; 16.4700x over previous
import functools

import jax
import jax.numpy as jnp
from jax import lax
from jax.experimental import pallas as pl
from jax.experimental.pallas import tpu as pltpu
from jax.experimental.pallas import tpu_sc as plsc

N = 10000
E = 320000
D = 128
H = 128
C = 40
CP = 64

NC = 2
NS = 16
NW = NC * NS
CH = 80
EPW = E // NW
NCHUNK = EPW // CH
NP = 10240
RPT = NP // NS

@functools.cache
def _sc_kernels():
    mesh = plsc.VectorSubcoreMesh(core_axis_name="c", subcore_axis_name="s",
                                  num_cores=NC, num_subcores=NS)

    @functools.partial(
        pl.kernel,
        out_type=[
            jax.ShapeDtypeStruct((NC, NP, 8), jnp.float32),
            jax.ShapeDtypeStruct((NC, NP, 8), jnp.float32),
        ],
        mesh=mesh,
        compiler_params=pltpu.CompilerParams(use_tc_tiling_on_sc=False),
        scratch_types=[
            pltpu.VMEM((NCHUNK, CH), jnp.int32),
            pltpu.VMEM((NCHUNK, CH), jnp.int32),
            pltpu.VMEM((CH, 8), jnp.float32),
            pltpu.VMEM_SHARED((NP, 8), jnp.float32),
            pltpu.VMEM_SHARED((NP, 8), jnp.float32),
        ],
    )
    def deg_kernel(dst1_hbm, dst2_hbm, ones_hbm, zeros_hbm, out1_hbm, out2_hbm,
                   d1_v, d2_v, ones_v, acc1_sh, acc2_sh):
        c = lax.axis_index("c")
        s = lax.axis_index("s")
        wid = s * NC + c
        pltpu.sync_copy(zeros_hbm.at[pl.ds(s * RPT, RPT)], acc1_sh.at[pl.ds(s * RPT, RPT)])
        pltpu.sync_copy(zeros_hbm.at[pl.ds(s * RPT, RPT)], acc2_sh.at[pl.ds(s * RPT, RPT)])
        pltpu.sync_copy(dst1_hbm.at[wid], d1_v)
        pltpu.sync_copy(dst2_hbm.at[wid], d2_v)
        pltpu.sync_copy(ones_hbm, ones_v)
        plsc.subcore_barrier()

        def body(j, carry):
            pltpu.sync_copy(ones_v, acc1_sh.at[d1_v.at[j]], add=True)
            pltpu.sync_copy(ones_v, acc2_sh.at[d2_v.at[j]], add=True)
            return carry

        lax.fori_loop(0, NCHUNK, body, 0)
        plsc.subcore_barrier()
        pltpu.sync_copy(acc1_sh.at[pl.ds(s * RPT, RPT)], out1_hbm.at[c, pl.ds(s * RPT, RPT)])
        pltpu.sync_copy(acc2_sh.at[pl.ds(s * RPT, RPT)], out2_hbm.at[c, pl.ds(s * RPT, RPT)])

    def make_agg(F):
        @functools.partial(
            pl.kernel,
            out_type=jax.ShapeDtypeStruct((NC, NP, F), jnp.float32),
            mesh=mesh,
            compiler_params=pltpu.CompilerParams(use_tc_tiling_on_sc=False),
            scratch_types=[
                pltpu.VMEM((NCHUNK, CH), jnp.int32),
                pltpu.VMEM((NCHUNK, CH), jnp.int32),
                pltpu.VMEM((CH, F), jnp.float32),
                pltpu.VMEM_SHARED((NP, F), jnp.float32),
                pltpu.SemaphoreType.DMA,
            ],
        )
        def agg(g_hbm, src_hbm, dst_hbm, zeros_hbm, out_hbm, src_v, dst_v, buf_v,
                acc_sh, sem):
            c = lax.axis_index("c")
            s = lax.axis_index("s")
            wid = s * NC + c
            pltpu.sync_copy(zeros_hbm.at[pl.ds(s * RPT, RPT)], acc_sh.at[pl.ds(s * RPT, RPT)])
            pltpu.sync_copy(src_hbm.at[wid], src_v)
            pltpu.sync_copy(dst_hbm.at[wid], dst_v)
            plsc.subcore_barrier()

            def body(j, carry):
                pltpu.async_copy(g_hbm.at[src_v.at[j]], buf_v, sem).wait()
                pltpu.sync_copy(buf_v, acc_sh.at[dst_v.at[j]], add=True)
                return carry

            lax.fori_loop(0, NCHUNK, body, 0)
            plsc.subcore_barrier()
            pltpu.sync_copy(acc_sh.at[pl.ds(s * RPT, RPT)], out_hbm.at[c, pl.ds(s * RPT, RPT)])

        return agg

    return deg_kernel, make_agg(H), make_agg(CP)


def _stage_b_body(x1_ref, x2_ref, w1_ref, degcat_ref, g1_ref, g2_ref):
    dc = degcat_ref[...]
    di1 = lax.rsqrt(dc[:, 0:1] + dc[:, 8:9] + 1.0)
    di2 = lax.rsqrt(dc[:, 16:17] + dc[:, 24:25] + 1.0)
    w1 = w1_ref[...]
    g1_ref[...] = jnp.dot(x1_ref[...], w1, preferred_element_type=jnp.float32) * di1
    g2_ref[...] = jnp.dot(x2_ref[...], w1, preferred_element_type=jnp.float32) * di2


_BR = 1000


def _stage_b(x1, x2, W1, degcat):
    grid = N // _BR
    return pl.pallas_call(
        _stage_b_body,
        grid=(grid,),
        in_specs=[
            pl.BlockSpec((_BR, D), lambda i: (i, 0)),
            pl.BlockSpec((_BR, D), lambda i: (i, 0)),
            pl.BlockSpec((D, H), lambda i: (0, 0)),
            pl.BlockSpec((_BR, 32), lambda i: (i, 0)),
        ],
        out_specs=[
            pl.BlockSpec((_BR, H), lambda i: (i, 0)),
            pl.BlockSpec((_BR, H), lambda i: (i, 0)),
        ],
        out_shape=[
            jax.ShapeDtypeStruct((N, H), jnp.float32),
            jax.ShapeDtypeStruct((N, H), jnp.float32),
        ],
    )(x1, x2, W1, degcat)


def _stage_d_body(g1_ref, a10_ref, a11_ref, g2_ref, a20_ref, a21_ref,
                  w2_ref, b1_ref, degcat_ref, gy1_ref, gy2_ref):
    dc = degcat_ref[...]
    di1 = lax.rsqrt(dc[:, 0:1] + dc[:, 8:9] + 1.0)
    di2 = lax.rsqrt(dc[:, 16:17] + dc[:, 24:25] + 1.0)
    w2 = w2_ref[...]
    b1 = b1_ref[...]
    h1 = jnp.maximum(di1 * (a10_ref[...] + a11_ref[...] + g1_ref[...]) + b1, 0.0)
    h2 = jnp.maximum(di2 * (a20_ref[...] + a21_ref[...] + g2_ref[...]) + b1, 0.0)
    gy1_ref[...] = jnp.dot(h1, w2, preferred_element_type=jnp.float32) * di1
    gy2_ref[...] = jnp.dot(h2, w2, preferred_element_type=jnp.float32) * di2


def _stage_d(g1, a10, a11, g2, a20, a21, W2p, b1r, degcat):
    grid = N // _BR
    row = lambda i: (i, 0)
    fixed = lambda i: (0, 0)
    return pl.pallas_call(
        _stage_d_body,
        grid=(grid,),
        in_specs=[
            pl.BlockSpec((_BR, H), row),
            pl.BlockSpec((_BR, H), row),
            pl.BlockSpec((_BR, H), row),
            pl.BlockSpec((_BR, H), row),
            pl.BlockSpec((_BR, H), row),
            pl.BlockSpec((_BR, H), row),
            pl.BlockSpec((H, CP), fixed),
            pl.BlockSpec((1, H), fixed),
            pl.BlockSpec((_BR, 32), row),
        ],
        out_specs=[
            pl.BlockSpec((_BR, CP), row),
            pl.BlockSpec((_BR, CP), row),
        ],
        out_shape=[
            jax.ShapeDtypeStruct((N, CP), jnp.float32),
            jax.ShapeDtypeStruct((N, CP), jnp.float32),
        ],
    )(g1, a10, a11, g2, a20, a21, W2p, b1r, degcat)


def _stage_f_body(gy1_ref, ay10_ref, ay11_ref, gy2_ref, ay20_ref, ay21_ref,
                  b2_ref, degcat_ref, ly_ref, lz_ref, cd_ref):
    dc = degcat_ref[...]
    di1 = lax.rsqrt(dc[:, 0:1] + dc[:, 8:9] + 1.0)
    di2 = lax.rsqrt(dc[:, 16:17] + dc[:, 24:25] + 1.0)
    b2 = b2_ref[...]
    y = di1 * (ay10_ref[...] + ay11_ref[...] + gy1_ref[...]) + b2
    z = di2 * (ay20_ref[...] + ay21_ref[...] + gy2_ref[...]) + b2
    ny = jnp.maximum(jnp.sqrt(jnp.sum(y * y, axis=1, keepdims=True)), 1e-8)
    nz = jnp.maximum(jnp.sqrt(jnp.sum(z * z, axis=1, keepdims=True)), 1e-8)
    cos = jnp.sum(y * z, axis=1, keepdims=True) / (ny * nz)
    cd_ref[...] = 1.0 - cos
    mask = lax.broadcasted_iota(jnp.int32, (_BR, CP), 1) < C
    neg = jnp.float32(-1e30)
    my = jnp.max(jnp.where(mask, y, neg), axis=1, keepdims=True)
    mz = jnp.max(jnp.where(mask, z, neg), axis=1, keepdims=True)
    lse_y = jnp.log(jnp.sum(jnp.where(mask, jnp.exp(y - my), 0.0), axis=1, keepdims=True))
    lse_z = jnp.log(jnp.sum(jnp.where(mask, jnp.exp(z - mz), 0.0), axis=1, keepdims=True))
    ly_ref[...] = y - my - lse_y
    lz_ref[...] = z - mz - lse_z


def _stage_f(gy1, ay10, ay11, gy2, ay20, ay21, b2p, degcat):
    grid = N // _BR
    row = lambda i: (i, 0)
    fixed = lambda i: (0, 0)
    return pl.pallas_call(
        _stage_f_body,
        grid=(grid,),
        in_specs=[
            pl.BlockSpec((_BR, CP), row),
            pl.BlockSpec((_BR, CP), row),
            pl.BlockSpec((_BR, CP), row),
            pl.BlockSpec((_BR, CP), row),
            pl.BlockSpec((_BR, CP), row),
            pl.BlockSpec((_BR, CP), row),
            pl.BlockSpec((1, CP), fixed),
            pl.BlockSpec((_BR, 32), row),
        ],
        out_specs=[
            pl.BlockSpec((_BR, CP), row),
            pl.BlockSpec((_BR, CP), row),
            pl.BlockSpec((_BR, 1), row),
        ],
        out_shape=[
            jax.ShapeDtypeStruct((N, CP), jnp.float32),
            jax.ShapeDtypeStruct((N, CP), jnp.float32),
            jax.ShapeDtypeStruct((N, 1), jnp.float32),
        ],
    )(gy1, ay10, ay11, gy2, ay20, ay21, b2p, degcat)


def kernel(x1, edge_index1, x2, edge_index2, W1, b1, W2, b2):
    src1 = edge_index1[0].reshape(NW, NCHUNK, CH)
    dst1 = edge_index1[1].reshape(NW, NCHUNK, CH)
    src2 = edge_index2[0].reshape(NW, NCHUNK, CH)
    dst2 = edge_index2[1].reshape(NW, NCHUNK, CH)

    ones8 = jnp.ones((CH, 8), jnp.float32)
    zeros8 = jnp.zeros((NP, 8), jnp.float32)
    zeros_h = jnp.zeros((NP, H), jnp.float32)
    zeros_c = jnp.zeros((NP, CP), jnp.float32)

    W2p = jnp.zeros((H, CP), jnp.float32).at[:, :C].set(W2)
    b1r = b1.reshape(1, H)
    b2p = jnp.zeros((1, CP), jnp.float32).at[0, :C].set(b2)

    _deg_kernel, _agg_h, _agg_c = _sc_kernels()

    deg1p, deg2p = _deg_kernel(dst1, dst2, ones8, zeros8)
    degcat = jnp.concatenate([deg1p[0, :N], deg1p[1, :N], deg2p[0, :N], deg2p[1, :N]], axis=1)

    g1, g2 = _stage_b(x1, x2, W1, degcat)
    a1 = _agg_h(g1, src1, dst1, zeros_h)
    a2 = _agg_h(g2, src2, dst2, zeros_h)
    gy1, gy2 = _stage_d(g1, a1[0, :N], a1[1, :N], g2, a2[0, :N], a2[1, :N], W2p, b1r, degcat)
    ay1 = _agg_c(gy1, src1, dst1, zeros_c)
    ay2 = _agg_c(gy2, src2, dst2, zeros_c)
    ly64, lz64, cd = _stage_f(gy1, ay1[0, :N], ay1[1, :N], gy2, ay2[0, :N], ay2[1, :N], b2p, degcat)

    ly = ly64[:, :C]
    lz = lz64[:, :C]
    return (ly, cd[:, 0], lz, ly, ly)
